# baseline (speedup 1.0000x reference)
import functools

import jax
import jax.numpy as jnp
from jax import lax
from jax.experimental import pallas as pl
from jax.experimental.pallas import tpu as pltpu
from jax.experimental.pallas import tpu_sc as plsc

N_NODES = 10000
D = 128
NPAD = 10240
E = 320000
NC, NS = 2, 16
NW = NC * NS
EB = 64
NB = 160
CHUNK = 8
NCHUNK = NB // CHUNK
NBUF = 5
EPT_PAD = NB * EB
EPAD = NW * EPT_PAD
T = 4096
TPT = T // NS
ROWS_PT = NPAD // NS


def _deg_kernel(e5):
    mesh = plsc.VectorSubcoreMesh(core_axis_name="c", subcore_axis_name="s")

    @functools.partial(
        pl.kernel,
        out_type=jax.ShapeDtypeStruct((NC, NPAD), jnp.float32),
        mesh=mesh,
        scratch_types=[
            pltpu.VMEM((NCHUNK, CHUNK, EB), jnp.int32),
            pltpu.VMEM((EB,), jnp.float32),
            pltpu.VMEM((ROWS_PT,), jnp.float32),
            pltpu.VMEM_SHARED((NPAD,), jnp.float32),
            pltpu.SemaphoreType.DMA,
        ],
    )
    def k(e_hbm, degp_hbm, dst_v, ones_v, outv, deg_sh, ssem):
        c = lax.axis_index("c")
        s = lax.axis_index("s")
        wid = c * NS + s
        z16 = jnp.zeros((16,), jnp.float32)
        o16 = jnp.ones((16,), jnp.float32)

        @pl.loop(0, ROWS_PT // 16)
        def _(i):
            outv[pl.ds(i * 16, 16)] = z16

        @pl.loop(0, EB // 16)
        def _(i):
            ones_v[pl.ds(i * 16, 16)] = o16

        pltpu.sync_copy(e_hbm.at[1, wid], dst_v)
        pltpu.sync_copy(outv, deg_sh.at[pl.ds(s * ROWS_PT, ROWS_PT)])
        plsc.subcore_barrier()

        @pl.loop(0, NCHUNK)
        def _(cc):
            for b in range(CHUNK):
                pltpu.async_copy(ones_v, deg_sh.at[dst_v.at[cc, b]], ssem,
                                 add=True)
            for b in range(CHUNK):
                pltpu.make_async_copy(ones_v, deg_sh.at[dst_v.at[cc, b]],
                                      ssem).wait()

        plsc.subcore_barrier()
        pltpu.sync_copy(deg_sh.at[pl.ds(s * ROWS_PT, ROWS_PT)], outv)
        pltpu.sync_copy(outv, degp_hbm.at[c, pl.ds(s * ROWS_PT, ROWS_PT)])

    return k(e5)


def _xw_kernel(x, W_gcn, degp):
    BLK = 1024

    def body(x_ref, w_ref, degp_ref, y_ref, dinv_ref):
        i = pl.program_id(0)
        xw = jnp.dot(x_ref[...], w_ref[...], preferred_element_type=jnp.float32)
        ones2 = jnp.ones((NC, 1), jnp.float32)
        deg = lax.dot_general(degp_ref[...], ones2,
                              (((0,), (0,)), ((), ())),
                              preferred_element_type=jnp.float32)
        dinv = lax.rsqrt(deg + 1.0)
        row = i * BLK + lax.broadcasted_iota(jnp.int32, (BLK, 1), 0)
        y_ref[...] = jnp.where(row < N_NODES, xw * dinv, 0.0)
        dinv_ref[...] = dinv.reshape(BLK)

    return pl.pallas_call(
        body,
        grid=(NPAD // BLK,),
        in_specs=[
            pl.BlockSpec((BLK, D), lambda i: (i, 0)),
            pl.BlockSpec((D, D), lambda i: (0, 0)),
            pl.BlockSpec((NC, BLK), lambda i: (0, i)),
        ],
        out_specs=[
            pl.BlockSpec((BLK, D), lambda i: (i, 0)),
            pl.BlockSpec((BLK,), lambda i: (i,)),
        ],
        out_shape=[
            jax.ShapeDtypeStruct((NPAD, D), jnp.float32),
            jax.ShapeDtypeStruct((NPAD,), jnp.float32),
        ],
    )(x, W_gcn, degp)


def _edge_kernel(y, e5, tgt3d, dinv):
    mesh = plsc.VectorSubcoreMesh(core_axis_name="c", subcore_axis_name="s")

    @functools.partial(
        pl.kernel,
        out_type=[
            jax.ShapeDtypeStruct((NC, T, D), jnp.float32),
            jax.ShapeDtypeStruct((T, D), jnp.float32),
            jax.ShapeDtypeStruct((T,), jnp.float32),
        ],
        mesh=mesh,
        scratch_types=[
            pltpu.VMEM((2, CHUNK, EB), jnp.int32),
            pltpu.VMEM((2, CHUNK, EB), jnp.int32),
            pltpu.VMEM((2, 128), jnp.int32),
            pltpu.VMEM((NBUF, EB, D), jnp.float32),
            pltpu.VMEM((TPT,), jnp.float32),
            pltpu.VMEM_SHARED((NPAD, D), jnp.float32),
            pltpu.SemaphoreType.DMA,
            pltpu.SemaphoreType.DMA,
            pltpu.SemaphoreType.DMA,
        ],
    )
    def k(y_hbm, e_hbm, tgt_hbm, dinv_hbm,
          p_hbm, yt_hbm, dt_hbm,
          srcb, dstb, tgt_v, buf, dt_v, acc_sh, gsem, ssem, isem):
        c = lax.axis_index("c")
        s = lax.axis_index("s")
        wid = c * NS + s
        z16 = jnp.zeros((16,), jnp.float32)

        pltpu.sync_copy(e_hbm.at[0, wid, 0], srcb.at[0])
        pltpu.sync_copy(e_hbm.at[1, wid, 0], dstb.at[0])
        pltpu.sync_copy(tgt_hbm.at[s], tgt_v)

        @pl.loop(0, EB)
        def _(r):
            for j in range(D // 16):
                buf[0, r, pl.ds(j * 16, 16)] = z16

        for kk in range(ROWS_PT // EB):
            pltpu.sync_copy(buf.at[0], acc_sh.at[pl.ds(s * ROWS_PT + kk * EB, EB)])
        plsc.subcore_barrier()

        def start_gather(p, b, i):
            pltpu.async_copy(y_hbm.at[srcb.at[p, b]], buf.at[i], gsem)

        def wait_gather(p, b, i):
            pltpu.make_async_copy(y_hbm.at[srcb.at[p, b]], buf.at[i],
                                  gsem).wait()

        def start_scatter(p, b, i):
            pltpu.async_copy(buf.at[i], acc_sh.at[dstb.at[p, b]], ssem,
                             add=True)

        def wait_scatter(p, b, i):
            pltpu.make_async_copy(buf.at[i], acc_sh.at[dstb.at[p, b]],
                                  ssem).wait()

        for b in range(NBUF - 1):
            start_gather(0, b, b)

        @pl.loop(0, NCHUNK)
        def _(cc):
            p = lax.rem(cc, 2)
            pn = lax.rem(cc + 1, 2)
            g0 = cc * CHUNK

            def ib(off):
                return lax.rem(g0 + off, NBUF)

            for b in range(CHUNK):
                wait_gather(p, b, ib(b))
                start_scatter(p, b, ib(b))

                if b == 0:
                    @pl.when(cc >= 1)
                    def _():
                        wait_scatter(pn, CHUNK - 1, ib(-1))
                else:
                    wait_scatter(p, b - 1, ib(b - 1))

                if b == 0:
                    @pl.when(cc + 1 < NCHUNK)
                    def _():
                        pltpu.async_copy(e_hbm.at[0, wid, cc + 1], srcb.at[pn],
                                         isem)
                        pltpu.async_copy(e_hbm.at[1, wid, cc + 1], dstb.at[pn],
                                         isem)
                if b == 3:
                    @pl.when(cc + 1 < NCHUNK)
                    def _():
                        pltpu.make_async_copy(e_hbm.at[0, wid, 0], srcb.at[pn],
                                              isem).wait()
                        pltpu.make_async_copy(e_hbm.at[1, wid, 0], dstb.at[pn],
                                              isem).wait()

                bn = b + NBUF - 1
                if bn < CHUNK:
                    start_gather(p, bn, ib(bn))
                else:
                    @pl.when(cc + 1 < NCHUNK)
                    def _():
                        start_gather(pn, bn - CHUNK, ib(bn))

        wait_scatter((NCHUNK - 1) % 2, CHUNK - 1, (NB - 1) % NBUF)
        plsc.subcore_barrier()

        GB = 64

        def round_idx(r):
            if r < 4:
                return tgt_v.at[r // 2, pl.ds((r % 2) * GB, GB)]
            q = r - 4
            return tgt_v.at[c, pl.ds(q * GB, GB)]

        def round_table(r):
            return acc_sh if r < 4 else y_hbm

        def round_out(r):
            if r < 4:
                return p_hbm.at[c, pl.ds(s * TPT + r * GB, GB)]
            q = r - 4
            return yt_hbm.at[pl.ds(s * TPT + c * 128 + q * GB, GB)]

        for r in range(6):
            pltpu.async_copy(round_table(r).at[round_idx(r)],
                             buf.at[r % NBUF], gsem).wait()
            pltpu.sync_copy(buf.at[r % NBUF], round_out(r))

        @pl.when(c == 0)
        def _():
            for j in range(2):
                pltpu.async_copy(dinv_hbm.at[tgt_v.at[j]],
                                 dt_v.at[pl.ds(j * 128, 128)], gsem).wait()
            pltpu.sync_copy(dt_v, dt_hbm.at[pl.ds(s * TPT, TPT)])

    return k(y, e5, tgt3d, dinv)


def _mlp_kernel(truck_x, P, yt, dt, W_truck, b_truck, b_gcn,
                W_comb, b_comb, W_out, b_out):
    BLK = 1024
    NCLS = 64

    def body(tx_ref, p_ref, yt_ref, dt_ref, wt_ref, bt_ref, bg_ref,
             wc_ref, bc_ref, wo_ref, bo_ref, out_ref):
        te = jnp.maximum(
            jnp.dot(tx_ref[...], wt_ref[...], preferred_element_type=jnp.float32)
            + bt_ref[...], 0.0)
        pp = p_ref[...]
        dt_col = dt_ref[...].reshape(BLK, 1)
        node = jnp.maximum(
            dt_col * (pp[0] + pp[1] + yt_ref[...]) + bg_ref[...], 0.0)
        wc = wc_ref[...]
        h = jnp.maximum(
            jnp.dot(te, wc[:D], preferred_element_type=jnp.float32)
            + jnp.dot(node, wc[D:], preferred_element_type=jnp.float32)
            + bc_ref[...], 0.0)
        out_ref[...] = (
            jnp.dot(h, wo_ref[...], preferred_element_type=jnp.float32)
            + bo_ref[...])

    full = lambda shape: pl.BlockSpec(shape, lambda i: tuple(0 for _ in shape))
    return pl.pallas_call(
        body,
        grid=(T // BLK,),
        in_specs=[
            pl.BlockSpec((BLK, 32), lambda i: (i, 0)),
            pl.BlockSpec((NC, BLK, D), lambda i: (0, i, 0)),
            pl.BlockSpec((BLK, D), lambda i: (i, 0)),
            pl.BlockSpec((BLK,), lambda i: (i,)),
            full((32, D)), full((1, D)), full((1, D)),
            full((2 * D, D)), full((1, D)),
            full((D, NCLS)), full((1, NCLS)),
        ],
        out_specs=pl.BlockSpec((BLK, NCLS), lambda i: (i, 0)),
        out_shape=jax.ShapeDtypeStruct((T, NCLS), jnp.float32),
    )(truck_x, P, yt, dt, W_truck, b_truck, b_gcn, W_comb, b_comb,
      W_out, b_out)


def kernel(x, edge_index, truck_x, cur_targets, W_gcn, b_gcn, W_truck, b_truck,
           W_comb, b_comb, W_out, b_out):
    ei = edge_index.astype(jnp.int32)
    pad_rows = N_NODES + (jnp.arange(EPAD - E, dtype=jnp.int32)
                          % (NPAD - N_NODES))
    pad2 = jnp.broadcast_to(pad_rows, (2, EPAD - E))
    e5 = jnp.concatenate([ei, pad2], axis=1).reshape(2, NW, NCHUNK, CHUNK, EB)
    tgt3d = cur_targets.astype(jnp.int32).reshape(NS, 2, 128)

    degp = _deg_kernel(e5)
    y, dinv = _xw_kernel(x, W_gcn, degp)
    P, yt, dt = _edge_kernel(y, e5, tgt3d, dinv)

    out = _mlp_kernel(
        truck_x, P, yt, dt,
        W_truck, b_truck.reshape(1, D), b_gcn.reshape(1, D),
        W_comb, b_comb.reshape(1, D),
        W_out, b_out.reshape(1, 64))
    return out


# speedup vs baseline: 1.0253x; 1.0253x over previous
import functools

import jax
import jax.numpy as jnp
from jax import lax
from jax.experimental import pallas as pl
from jax.experimental.pallas import tpu as pltpu
from jax.experimental.pallas import tpu_sc as plsc

N_NODES = 10000
D = 128
NPAD = 10240
E = 320000
NC, NS = 2, 16
NW = NC * NS
EB = 64
NB = 160
CHUNK = 8
NCHUNK = NB // CHUNK
NBUF = 5
EPT_PAD = NB * EB
EPAD = NW * EPT_PAD
T = 4096
TPT = T // NS
ROWS_PT = NPAD // NS


def _deg_kernel(e5):
    mesh = plsc.VectorSubcoreMesh(core_axis_name="c", subcore_axis_name="s")

    @functools.partial(
        pl.kernel,
        out_type=jax.ShapeDtypeStruct((NC, NPAD), jnp.float32),
        mesh=mesh,
        scratch_types=[
            pltpu.VMEM((NCHUNK, CHUNK, EB), jnp.int32),
            pltpu.VMEM((EB,), jnp.float32),
            pltpu.VMEM((ROWS_PT,), jnp.float32),
            pltpu.VMEM_SHARED((NPAD,), jnp.float32),
            pltpu.SemaphoreType.DMA,
        ],
    )
    def k(e_hbm, degp_hbm, dst_v, ones_v, outv, deg_sh, ssem):
        c = lax.axis_index("c")
        s = lax.axis_index("s")
        wid = c * NS + s
        z16 = jnp.zeros((16,), jnp.float32)
        o16 = jnp.ones((16,), jnp.float32)

        @pl.loop(0, ROWS_PT // 16)
        def _(i):
            outv[pl.ds(i * 16, 16)] = z16

        @pl.loop(0, EB // 16)
        def _(i):
            ones_v[pl.ds(i * 16, 16)] = o16

        pltpu.sync_copy(e_hbm.at[1, wid], dst_v)
        pltpu.sync_copy(outv, deg_sh.at[pl.ds(s * ROWS_PT, ROWS_PT)])
        plsc.subcore_barrier()

        @pl.loop(0, NCHUNK)
        def _(cc):
            for b in range(CHUNK):
                pltpu.async_copy(ones_v, deg_sh.at[dst_v.at[cc, b]], ssem,
                                 add=True)

                @pl.when(cc >= 1)
                def _():
                    pltpu.make_async_copy(ones_v, deg_sh.at[dst_v.at[cc, b]],
                                          ssem).wait()

        for b in range(CHUNK):
            pltpu.make_async_copy(ones_v, deg_sh.at[dst_v.at[0, b]],
                                  ssem).wait()

        plsc.subcore_barrier()
        pltpu.sync_copy(deg_sh.at[pl.ds(s * ROWS_PT, ROWS_PT)], outv)
        pltpu.sync_copy(outv, degp_hbm.at[c, pl.ds(s * ROWS_PT, ROWS_PT)])

    return k(e5)


def _xw_kernel(x, W_gcn, degp):
    BLK = 1024

    def body(x_ref, w_ref, degp_ref, y_ref, dinv_ref):
        i = pl.program_id(0)
        xw = jnp.dot(x_ref[...], w_ref[...], preferred_element_type=jnp.float32)
        ones2 = jnp.ones((NC, 1), jnp.float32)
        deg = lax.dot_general(degp_ref[...], ones2,
                              (((0,), (0,)), ((), ())),
                              preferred_element_type=jnp.float32)
        dinv = lax.rsqrt(deg + 1.0)
        row = i * BLK + lax.broadcasted_iota(jnp.int32, (BLK, 1), 0)
        y_ref[...] = jnp.where(row < N_NODES, xw * dinv, 0.0)
        dinv_ref[...] = dinv.reshape(BLK)

    return pl.pallas_call(
        body,
        grid=(NPAD // BLK,),
        in_specs=[
            pl.BlockSpec((BLK, D), lambda i: (i, 0)),
            pl.BlockSpec((D, D), lambda i: (0, 0)),
            pl.BlockSpec((NC, BLK), lambda i: (0, i)),
        ],
        out_specs=[
            pl.BlockSpec((BLK, D), lambda i: (i, 0)),
            pl.BlockSpec((BLK,), lambda i: (i,)),
        ],
        out_shape=[
            jax.ShapeDtypeStruct((NPAD, D), jnp.float32),
            jax.ShapeDtypeStruct((NPAD,), jnp.float32),
        ],
    )(x, W_gcn, degp)


def _edge_kernel(y, e5, tgt3d, dinv):
    mesh = plsc.VectorSubcoreMesh(core_axis_name="c", subcore_axis_name="s")

    @functools.partial(
        pl.kernel,
        out_type=[
            jax.ShapeDtypeStruct((NC, T, D), jnp.float32),
            jax.ShapeDtypeStruct((T, D), jnp.float32),
            jax.ShapeDtypeStruct((T,), jnp.float32),
        ],
        mesh=mesh,
        scratch_types=[
            pltpu.VMEM((2, CHUNK, EB), jnp.int32),
            pltpu.VMEM((2, CHUNK, EB), jnp.int32),
            pltpu.VMEM((2, 128), jnp.int32),
            pltpu.VMEM((NBUF, EB, D), jnp.float32),
            pltpu.VMEM((TPT,), jnp.float32),
            pltpu.VMEM_SHARED((NPAD, D), jnp.float32),
            pltpu.SemaphoreType.DMA,
            pltpu.SemaphoreType.DMA,
            pltpu.SemaphoreType.DMA,
        ],
    )
    def k(y_hbm, e_hbm, tgt_hbm, dinv_hbm,
          p_hbm, yt_hbm, dt_hbm,
          srcb, dstb, tgt_v, buf, dt_v, acc_sh, gsem, ssem, isem):
        c = lax.axis_index("c")
        s = lax.axis_index("s")
        wid = c * NS + s
        z16 = jnp.zeros((16,), jnp.float32)

        pltpu.sync_copy(e_hbm.at[0, wid, 0], srcb.at[0])
        pltpu.sync_copy(e_hbm.at[1, wid, 0], dstb.at[0])
        pltpu.sync_copy(tgt_hbm.at[s], tgt_v)

        @pl.loop(0, EB)
        def _(r):
            for j in range(D // 16):
                buf[0, r, pl.ds(j * 16, 16)] = z16

        for kk in range(ROWS_PT // EB):
            pltpu.sync_copy(buf.at[0], acc_sh.at[pl.ds(s * ROWS_PT + kk * EB, EB)])
        plsc.subcore_barrier()

        def start_gather(p, b, i):
            pltpu.async_copy(y_hbm.at[srcb.at[p, b]], buf.at[i], gsem)

        def wait_gather(p, b, i):
            pltpu.make_async_copy(y_hbm.at[srcb.at[p, b]], buf.at[i],
                                  gsem).wait()

        def start_scatter(p, b, i):
            pltpu.async_copy(buf.at[i], acc_sh.at[dstb.at[p, b]], ssem,
                             add=True)

        def wait_scatter(p, b, i):
            pltpu.make_async_copy(buf.at[i], acc_sh.at[dstb.at[p, b]],
                                  ssem).wait()

        for b in range(NBUF - 1):
            start_gather(0, b, b)

        @pl.loop(0, NCHUNK)
        def _(cc):
            p = lax.rem(cc, 2)
            pn = lax.rem(cc + 1, 2)
            g0 = cc * CHUNK

            def ib(off):
                return lax.rem(g0 + off, NBUF)

            for b in range(CHUNK):
                wait_gather(p, b, ib(b))
                start_scatter(p, b, ib(b))

                if b == 0:
                    @pl.when(cc >= 1)
                    def _():
                        wait_scatter(pn, CHUNK - 1, ib(-1))
                else:
                    wait_scatter(p, b - 1, ib(b - 1))

                if b == 0:
                    @pl.when(cc + 1 < NCHUNK)
                    def _():
                        pltpu.async_copy(e_hbm.at[0, wid, cc + 1], srcb.at[pn],
                                         isem)
                        pltpu.async_copy(e_hbm.at[1, wid, cc + 1], dstb.at[pn],
                                         isem)
                if b == 3:
                    @pl.when(cc + 1 < NCHUNK)
                    def _():
                        pltpu.make_async_copy(e_hbm.at[0, wid, 0], srcb.at[pn],
                                              isem).wait()
                        pltpu.make_async_copy(e_hbm.at[1, wid, 0], dstb.at[pn],
                                              isem).wait()

                bn = b + NBUF - 1
                if bn < CHUNK:
                    start_gather(p, bn, ib(bn))
                else:
                    @pl.when(cc + 1 < NCHUNK)
                    def _():
                        start_gather(pn, bn - CHUNK, ib(bn))

        wait_scatter((NCHUNK - 1) % 2, CHUNK - 1, (NB - 1) % NBUF)
        plsc.subcore_barrier()

        GB = 64

        def round_idx(r):
            if r < 4:
                return tgt_v.at[r // 2, pl.ds((r % 2) * GB, GB)]
            q = r - 4
            return tgt_v.at[c, pl.ds(q * GB, GB)]

        def round_table(r):
            return acc_sh if r < 4 else y_hbm

        def round_out(r):
            if r < 4:
                return p_hbm.at[c, pl.ds(s * TPT + r * GB, GB)]
            q = r - 4
            return yt_hbm.at[pl.ds(s * TPT + c * 128 + q * GB, GB)]

        for r in range(6):
            pltpu.async_copy(round_table(r).at[round_idx(r)],
                             buf.at[r % NBUF], gsem).wait()
            pltpu.sync_copy(buf.at[r % NBUF], round_out(r))

        @pl.when(c == 0)
        def _():
            for j in range(2):
                pltpu.async_copy(dinv_hbm.at[tgt_v.at[j]],
                                 dt_v.at[pl.ds(j * 128, 128)], gsem).wait()
            pltpu.sync_copy(dt_v, dt_hbm.at[pl.ds(s * TPT, TPT)])

    return k(y, e5, tgt3d, dinv)


def _mlp_kernel(truck_x, P, yt, dt, W_truck, b_truck, b_gcn,
                W_comb, b_comb, W_out, b_out):
    BLK = 1024
    NCLS = 64

    def body(tx_ref, p_ref, yt_ref, dt_ref, wt_ref, bt_ref, bg_ref,
             wc_ref, bc_ref, wo_ref, bo_ref, out_ref):
        te = jnp.maximum(
            jnp.dot(tx_ref[...], wt_ref[...], preferred_element_type=jnp.float32)
            + bt_ref[...], 0.0)
        pp = p_ref[...]
        dt_col = dt_ref[...].reshape(BLK, 1)
        node = jnp.maximum(
            dt_col * (pp[0] + pp[1] + yt_ref[...]) + bg_ref[...], 0.0)
        wc = wc_ref[...]
        h = jnp.maximum(
            jnp.dot(te, wc[:D], preferred_element_type=jnp.float32)
            + jnp.dot(node, wc[D:], preferred_element_type=jnp.float32)
            + bc_ref[...], 0.0)
        out_ref[...] = (
            lax.dot_general(wo_ref[...], h, (((0,), (1,)), ((), ())),
                            preferred_element_type=jnp.float32)
            + bo_ref[...])

    full = lambda shape: pl.BlockSpec(shape, lambda i: tuple(0 for _ in shape))
    return pl.pallas_call(
        body,
        grid=(T // BLK,),
        in_specs=[
            pl.BlockSpec((BLK, 32), lambda i: (i, 0)),
            pl.BlockSpec((NC, BLK, D), lambda i: (0, i, 0)),
            pl.BlockSpec((BLK, D), lambda i: (i, 0)),
            pl.BlockSpec((BLK,), lambda i: (i,)),
            full((32, D)), full((1, D)), full((1, D)),
            full((2 * D, D)), full((1, D)),
            full((D, NCLS)), full((NCLS, 1)),
        ],
        out_specs=pl.BlockSpec((NCLS, BLK), lambda i: (0, i)),
        out_shape=jax.ShapeDtypeStruct((NCLS, T), jnp.float32),
    )(truck_x, P, yt, dt, W_truck, b_truck, b_gcn, W_comb, b_comb,
      W_out, b_out)


def kernel(x, edge_index, truck_x, cur_targets, W_gcn, b_gcn, W_truck, b_truck,
           W_comb, b_comb, W_out, b_out):
    ei = edge_index.astype(jnp.int32)
    pad_rows = N_NODES + (jnp.arange(EPAD - E, dtype=jnp.int32)
                          % (NPAD - N_NODES))
    pad2 = jnp.broadcast_to(pad_rows, (2, EPAD - E))
    e5 = jnp.concatenate([ei, pad2], axis=1).reshape(2, NW, NCHUNK, CHUNK, EB)
    tgt3d = cur_targets.astype(jnp.int32).reshape(NS, 2, 128)

    degp = _deg_kernel(e5)
    y, dinv = _xw_kernel(x, W_gcn, degp)
    P, yt, dt = _edge_kernel(y, e5, tgt3d, dinv)

    out_t = _mlp_kernel(
        truck_x, P, yt, dt,
        W_truck, b_truck.reshape(1, D), b_gcn.reshape(1, D),
        W_comb, b_comb.reshape(1, D),
        W_out, b_out.reshape(64, 1))
    return out_t.T


# speedup vs baseline: 1.0496x; 1.0237x over previous
import functools

import jax
import jax.numpy as jnp
from jax import lax
from jax.experimental import pallas as pl
from jax.experimental.pallas import tpu as pltpu
from jax.experimental.pallas import tpu_sc as plsc

N_NODES = 10000
D = 128
NPAD = 10240
E = 320000
NC, NS = 2, 16
NW = NC * NS
EB = 64
NB = 160
CHUNK = 8
NCHUNK = NB // CHUNK
NBUF = 5
EPT_PAD = NB * EB
EPAD = NW * EPT_PAD
T = 4096
TPT = T // NS
ROWS_PT = NPAD // NS


def _deg_kernel(e5):
    mesh = plsc.VectorSubcoreMesh(core_axis_name="c", subcore_axis_name="s")

    @functools.partial(
        pl.kernel,
        out_type=jax.ShapeDtypeStruct((NC, NPAD), jnp.float32),
        mesh=mesh,
        scratch_types=[
            pltpu.VMEM((NCHUNK, CHUNK, EB), jnp.int32),
            pltpu.VMEM((EB,), jnp.float32),
            pltpu.VMEM((ROWS_PT,), jnp.float32),
            pltpu.VMEM_SHARED((NPAD,), jnp.float32),
            pltpu.SemaphoreType.DMA,
        ],
    )
    def k(e_hbm, degp_hbm, dst_v, ones_v, outv, deg_sh, ssem):
        c = lax.axis_index("c")
        s = lax.axis_index("s")
        wid = c * NS + s
        z16 = jnp.zeros((16,), jnp.float32)
        o16 = jnp.ones((16,), jnp.float32)

        @pl.loop(0, ROWS_PT // 16)
        def _(i):
            outv[pl.ds(i * 16, 16)] = z16

        @pl.loop(0, EB // 16)
        def _(i):
            ones_v[pl.ds(i * 16, 16)] = o16

        pltpu.sync_copy(e_hbm.at[1, wid], dst_v)
        pltpu.sync_copy(outv, deg_sh.at[pl.ds(s * ROWS_PT, ROWS_PT)])
        plsc.subcore_barrier()

        @pl.loop(0, NCHUNK)
        def _(cc):
            for b in range(CHUNK):
                pltpu.async_copy(ones_v, deg_sh.at[dst_v.at[cc, b]], ssem,
                                 add=True)

                @pl.when(cc >= 1)
                def _():
                    pltpu.make_async_copy(ones_v, deg_sh.at[dst_v.at[cc, b]],
                                          ssem).wait()

        for b in range(CHUNK):
            pltpu.make_async_copy(ones_v, deg_sh.at[dst_v.at[0, b]],
                                  ssem).wait()

        plsc.subcore_barrier()
        pltpu.sync_copy(deg_sh.at[pl.ds(s * ROWS_PT, ROWS_PT)], outv)
        pltpu.sync_copy(outv, degp_hbm.at[c, pl.ds(s * ROWS_PT, ROWS_PT)])

    return k(e5)


def _xw_kernel(x, W_gcn, degp):
    BLK = 2048

    def body(x_ref, w_ref, degp_ref, y_ref, dinv_ref):
        i = pl.program_id(0)
        xw = jnp.dot(x_ref[...], w_ref[...], preferred_element_type=jnp.float32)
        ones2 = jnp.ones((NC, 1), jnp.float32)
        deg = lax.dot_general(degp_ref[...], ones2,
                              (((0,), (0,)), ((), ())),
                              preferred_element_type=jnp.float32)
        dinv = lax.rsqrt(deg + 1.0)
        row = i * BLK + lax.broadcasted_iota(jnp.int32, (BLK, 1), 0)
        y_ref[...] = jnp.where(row < N_NODES, xw * dinv, 0.0)
        dinv_ref[...] = dinv.reshape(BLK)

    return pl.pallas_call(
        body,
        grid=(NPAD // BLK,),
        in_specs=[
            pl.BlockSpec((BLK, D), lambda i: (i, 0)),
            pl.BlockSpec((D, D), lambda i: (0, 0)),
            pl.BlockSpec((NC, BLK), lambda i: (0, i)),
        ],
        out_specs=[
            pl.BlockSpec((BLK, D), lambda i: (i, 0)),
            pl.BlockSpec((BLK,), lambda i: (i,)),
        ],
        out_shape=[
            jax.ShapeDtypeStruct((NPAD, D), jnp.float32),
            jax.ShapeDtypeStruct((NPAD,), jnp.float32),
        ],
    )(x, W_gcn, degp)


def _edge_kernel(y, e5, tgt3d, dinv):
    mesh = plsc.VectorSubcoreMesh(core_axis_name="c", subcore_axis_name="s")

    @functools.partial(
        pl.kernel,
        out_type=[
            jax.ShapeDtypeStruct((NC, T, D), jnp.float32),
            jax.ShapeDtypeStruct((T, D), jnp.float32),
            jax.ShapeDtypeStruct((T,), jnp.float32),
        ],
        mesh=mesh,
        scratch_types=[
            pltpu.VMEM((2, CHUNK, EB), jnp.int32),
            pltpu.VMEM((2, CHUNK, EB), jnp.int32),
            pltpu.VMEM((2, 128), jnp.int32),
            pltpu.VMEM((NBUF, EB, D), jnp.float32),
            pltpu.VMEM((TPT,), jnp.float32),
            pltpu.VMEM_SHARED((NPAD, D), jnp.float32),
            pltpu.SemaphoreType.DMA,
            pltpu.SemaphoreType.DMA,
            pltpu.SemaphoreType.DMA,
        ],
    )
    def k(y_hbm, e_hbm, tgt_hbm, dinv_hbm,
          p_hbm, yt_hbm, dt_hbm,
          srcb, dstb, tgt_v, buf, dt_v, acc_sh, gsem, ssem, isem):
        c = lax.axis_index("c")
        s = lax.axis_index("s")
        wid = c * NS + s
        z16 = jnp.zeros((16,), jnp.float32)

        pltpu.sync_copy(e_hbm.at[0, wid, 0], srcb.at[0])
        pltpu.sync_copy(e_hbm.at[1, wid, 0], dstb.at[0])
        pltpu.sync_copy(tgt_hbm.at[s], tgt_v)

        @pl.loop(0, EB)
        def _(r):
            for j in range(D // 16):
                buf[NBUF - 1, r, pl.ds(j * 16, 16)] = z16

        for kk in range(ROWS_PT // EB):
            pltpu.async_copy(buf.at[NBUF - 1],
                             acc_sh.at[pl.ds(s * ROWS_PT + kk * EB, EB)], isem)
        for kk in range(ROWS_PT // EB):
            pltpu.make_async_copy(buf.at[NBUF - 1],
                                  acc_sh.at[pl.ds(s * ROWS_PT, EB)],
                                  isem).wait()
        plsc.subcore_barrier()

        def start_gather(p, b, i):
            pltpu.async_copy(y_hbm.at[srcb.at[p, b]], buf.at[i], gsem)

        def wait_gather(p, b, i):
            pltpu.make_async_copy(y_hbm.at[srcb.at[p, b]], buf.at[i],
                                  gsem).wait()

        def start_scatter(p, b, i):
            pltpu.async_copy(buf.at[i], acc_sh.at[dstb.at[p, b]], ssem,
                             add=True)

        def wait_scatter(p, b, i):
            pltpu.make_async_copy(buf.at[i], acc_sh.at[dstb.at[p, b]],
                                  ssem).wait()

        for b in range(NBUF - 1):
            start_gather(0, b, b)

        @pl.loop(0, NCHUNK)
        def _(cc):
            p = lax.rem(cc, 2)
            pn = lax.rem(cc + 1, 2)
            g0 = cc * CHUNK

            def ib(off):
                return lax.rem(g0 + off, NBUF)

            for b in range(CHUNK):
                wait_gather(p, b, ib(b))
                start_scatter(p, b, ib(b))

                if b == 0:
                    @pl.when(cc >= 1)
                    def _():
                        wait_scatter(pn, CHUNK - 1, ib(-1))
                else:
                    wait_scatter(p, b - 1, ib(b - 1))

                if b == 0:
                    @pl.when(cc + 1 < NCHUNK)
                    def _():
                        pltpu.async_copy(e_hbm.at[0, wid, cc + 1], srcb.at[pn],
                                         isem)
                        pltpu.async_copy(e_hbm.at[1, wid, cc + 1], dstb.at[pn],
                                         isem)
                if b == 3:
                    @pl.when(cc + 1 < NCHUNK)
                    def _():
                        pltpu.make_async_copy(e_hbm.at[0, wid, 0], srcb.at[pn],
                                              isem).wait()
                        pltpu.make_async_copy(e_hbm.at[1, wid, 0], dstb.at[pn],
                                              isem).wait()

                bn = b + NBUF - 1
                if bn < CHUNK:
                    start_gather(p, bn, ib(bn))
                else:
                    @pl.when(cc + 1 < NCHUNK)
                    def _():
                        start_gather(pn, bn - CHUNK, ib(bn))

        wait_scatter((NCHUNK - 1) % 2, CHUNK - 1, (NB - 1) % NBUF)
        plsc.subcore_barrier()

        GB = 64

        def round_idx(r):
            if r < 4:
                return tgt_v.at[r // 2, pl.ds((r % 2) * GB, GB)]
            q = r - 4
            return tgt_v.at[c, pl.ds(q * GB, GB)]

        def round_table(r):
            return acc_sh if r < 4 else y_hbm

        def round_out(r):
            if r < 4:
                return p_hbm.at[c, pl.ds(s * TPT + r * GB, GB)]
            q = r - 4
            return yt_hbm.at[pl.ds(s * TPT + c * 128 + q * GB, GB)]

        for r in range(6):
            pltpu.async_copy(round_table(r).at[round_idx(r)],
                             buf.at[r % NBUF], gsem).wait()
            pltpu.sync_copy(buf.at[r % NBUF], round_out(r))

        @pl.when(c == 0)
        def _():
            for j in range(2):
                pltpu.async_copy(dinv_hbm.at[tgt_v.at[j]],
                                 dt_v.at[pl.ds(j * 128, 128)], gsem).wait()
            pltpu.sync_copy(dt_v, dt_hbm.at[pl.ds(s * TPT, TPT)])

    return k(y, e5, tgt3d, dinv)


def _mlp_kernel(truck_x, P, yt, dt, W_truck, b_truck, b_gcn,
                W_comb, b_comb, W_out, b_out):
    BLK = 1024
    NCLS = 64

    def body(tx_ref, p_ref, yt_ref, dt_ref, wt_ref, bt_ref, bg_ref,
             wc_ref, bc_ref, wo_ref, bo_ref, out_ref):
        te = jnp.maximum(
            jnp.dot(tx_ref[...], wt_ref[...], preferred_element_type=jnp.float32)
            + bt_ref[...], 0.0)
        pp = p_ref[...]
        dt_col = dt_ref[...].reshape(BLK, 1)
        node = jnp.maximum(
            dt_col * (pp[0] + pp[1] + yt_ref[...]) + bg_ref[...], 0.0)
        wc = wc_ref[...]
        h = jnp.maximum(
            jnp.dot(te, wc[:D], preferred_element_type=jnp.float32)
            + jnp.dot(node, wc[D:], preferred_element_type=jnp.float32)
            + bc_ref[...], 0.0)
        out_ref[...] = (
            lax.dot_general(wo_ref[...], h, (((0,), (1,)), ((), ())),
                            preferred_element_type=jnp.float32)
            + bo_ref[...])

    full = lambda shape: pl.BlockSpec(shape, lambda i: tuple(0 for _ in shape))
    return pl.pallas_call(
        body,
        grid=(T // BLK,),
        in_specs=[
            pl.BlockSpec((BLK, 32), lambda i: (i, 0)),
            pl.BlockSpec((NC, BLK, D), lambda i: (0, i, 0)),
            pl.BlockSpec((BLK, D), lambda i: (i, 0)),
            pl.BlockSpec((BLK,), lambda i: (i,)),
            full((32, D)), full((1, D)), full((1, D)),
            full((2 * D, D)), full((1, D)),
            full((D, NCLS)), full((NCLS, 1)),
        ],
        out_specs=pl.BlockSpec((NCLS, BLK), lambda i: (0, i)),
        out_shape=jax.ShapeDtypeStruct((NCLS, T), jnp.float32),
    )(truck_x, P, yt, dt, W_truck, b_truck, b_gcn, W_comb, b_comb,
      W_out, b_out)


def kernel(x, edge_index, truck_x, cur_targets, W_gcn, b_gcn, W_truck, b_truck,
           W_comb, b_comb, W_out, b_out):
    ei = edge_index.astype(jnp.int32)
    pad_rows = N_NODES + (jnp.arange(EPAD - E, dtype=jnp.int32)
                          % (NPAD - N_NODES))
    pad2 = jnp.broadcast_to(pad_rows, (2, EPAD - E))
    e5 = jnp.concatenate([ei, pad2], axis=1).reshape(2, NW, NCHUNK, CHUNK, EB)
    tgt3d = cur_targets.astype(jnp.int32).reshape(NS, 2, 128)

    degp = _deg_kernel(e5)
    y, dinv = _xw_kernel(x, W_gcn, degp)
    P, yt, dt = _edge_kernel(y, e5, tgt3d, dinv)

    out_t = _mlp_kernel(
        truck_x, P, yt, dt,
        W_truck, b_truck.reshape(1, D), b_gcn.reshape(1, D),
        W_comb, b_comb.reshape(1, D),
        W_out, b_out.reshape(64, 1))
    return out_t.T


# speedup vs baseline: 1.0642x; 1.0139x over previous
import functools

import jax
import jax.numpy as jnp
from jax import lax
from jax.experimental import pallas as pl
from jax.experimental.pallas import tpu as pltpu
from jax.experimental.pallas import tpu_sc as plsc

N_NODES = 10000
D = 128
NPAD = 10240
E = 320000
NC, NS = 2, 16
NW = NC * NS
EB = 64
NB = 160
CHUNK = 8
NCHUNK = NB // CHUNK
NBUF = 5
EPT_PAD = NB * EB
EPAD = NW * EPT_PAD
T = 4096
TPT = T // NS
ROWS_PT = NPAD // NS


def _deg_kernel(e5):
    mesh = plsc.VectorSubcoreMesh(core_axis_name="c", subcore_axis_name="s")

    @functools.partial(
        pl.kernel,
        out_type=jax.ShapeDtypeStruct((NC, NPAD), jnp.float32),
        mesh=mesh,
        scratch_types=[
            pltpu.VMEM((NCHUNK, CHUNK, EB), jnp.int32),
            pltpu.VMEM((EB,), jnp.float32),
            pltpu.VMEM((ROWS_PT,), jnp.float32),
            pltpu.VMEM_SHARED((NPAD,), jnp.float32),
            pltpu.SemaphoreType.DMA,
        ],
    )
    def k(e_hbm, degp_hbm, dst_v, ones_v, outv, deg_sh, ssem):
        c = lax.axis_index("c")
        s = lax.axis_index("s")
        wid = c * NS + s
        z16 = jnp.zeros((16,), jnp.float32)
        o16 = jnp.ones((16,), jnp.float32)

        @pl.loop(0, ROWS_PT // 16)
        def _(i):
            outv[pl.ds(i * 16, 16)] = z16

        @pl.loop(0, EB // 16)
        def _(i):
            ones_v[pl.ds(i * 16, 16)] = o16

        pltpu.sync_copy(e_hbm.at[1, wid], dst_v)
        pltpu.sync_copy(outv, deg_sh.at[pl.ds(s * ROWS_PT, ROWS_PT)])
        plsc.subcore_barrier()

        @pl.loop(0, NCHUNK)
        def _(cc):
            for b in range(CHUNK):
                pltpu.async_copy(ones_v, deg_sh.at[dst_v.at[cc, b]], ssem,
                                 add=True)

                @pl.when(cc >= 1)
                def _():
                    pltpu.make_async_copy(ones_v, deg_sh.at[dst_v.at[cc, b]],
                                          ssem).wait()

        for b in range(CHUNK):
            pltpu.make_async_copy(ones_v, deg_sh.at[dst_v.at[0, b]],
                                  ssem).wait()

        plsc.subcore_barrier()
        pltpu.sync_copy(deg_sh.at[pl.ds(s * ROWS_PT, ROWS_PT)], outv)
        pltpu.sync_copy(outv, degp_hbm.at[c, pl.ds(s * ROWS_PT, ROWS_PT)])

    return k(e5)


def _xw_kernel(x, W_gcn, degp):
    BLK = 2048

    def body(x_ref, w_ref, degp_ref, y_ref, dinv_ref):
        i = pl.program_id(0)
        xw = jnp.dot(x_ref[...], w_ref[...], preferred_element_type=jnp.float32)
        ones2 = jnp.ones((NC, 1), jnp.float32)
        deg = lax.dot_general(degp_ref[...], ones2,
                              (((0,), (0,)), ((), ())),
                              preferred_element_type=jnp.float32)
        dinv = lax.rsqrt(deg + 1.0)
        row = i * BLK + lax.broadcasted_iota(jnp.int32, (BLK, 1), 0)
        y_ref[...] = jnp.where(row < N_NODES, xw * dinv, 0.0)
        dinv_ref[...] = dinv.reshape(BLK)

    return pl.pallas_call(
        body,
        grid=(NPAD // BLK,),
        in_specs=[
            pl.BlockSpec((BLK, D), lambda i: (i, 0)),
            pl.BlockSpec((D, D), lambda i: (0, 0)),
            pl.BlockSpec((NC, BLK), lambda i: (0, i)),
        ],
        out_specs=[
            pl.BlockSpec((BLK, D), lambda i: (i, 0)),
            pl.BlockSpec((BLK,), lambda i: (i,)),
        ],
        out_shape=[
            jax.ShapeDtypeStruct((NPAD, D), jnp.float32),
            jax.ShapeDtypeStruct((NPAD,), jnp.float32),
        ],
    )(x, W_gcn, degp)


def _edge_kernel(y, e5, tgt3d, dinv):
    mesh = plsc.VectorSubcoreMesh(core_axis_name="c", subcore_axis_name="s")

    @functools.partial(
        pl.kernel,
        out_type=[
            jax.ShapeDtypeStruct((NC, T, D), jnp.float32),
            jax.ShapeDtypeStruct((T, D), jnp.float32),
            jax.ShapeDtypeStruct((T,), jnp.float32),
        ],
        mesh=mesh,
        scratch_types=[
            pltpu.VMEM((2, CHUNK, EB), jnp.int32),
            pltpu.VMEM((2, CHUNK, EB), jnp.int32),
            pltpu.VMEM((2, 128), jnp.int32),
            pltpu.VMEM((NBUF, EB, D), jnp.float32),
            pltpu.VMEM((TPT,), jnp.float32),
            pltpu.VMEM_SHARED((NPAD, D), jnp.float32),
            pltpu.SemaphoreType.DMA,
            pltpu.SemaphoreType.DMA,
            pltpu.SemaphoreType.DMA,
        ],
    )
    def k(y_hbm, e_hbm, tgt_hbm, dinv_hbm,
          p_hbm, yt_hbm, dt_hbm,
          srcb, dstb, tgt_v, buf, dt_v, acc_sh, gsem, ssem, isem):
        c = lax.axis_index("c")
        s = lax.axis_index("s")
        wid = c * NS + s
        z16 = jnp.zeros((16,), jnp.float32)

        pltpu.sync_copy(e_hbm.at[0, wid, 0], srcb.at[0])
        pltpu.sync_copy(e_hbm.at[1, wid, 0], dstb.at[0])
        pltpu.sync_copy(tgt_hbm.at[s], tgt_v)

        for b in range(NBUF - 1):
            pltpu.async_copy(y_hbm.at[srcb.at[0, b]], buf.at[b], gsem)

        @pl.loop(0, EB)
        def _(r):
            for j in range(D // 16):
                buf[NBUF - 1, r, pl.ds(j * 16, 16)] = z16

        for kk in range(ROWS_PT // EB):
            pltpu.async_copy(buf.at[NBUF - 1],
                             acc_sh.at[pl.ds(s * ROWS_PT + kk * EB, EB)], isem)
        for kk in range(ROWS_PT // EB):
            pltpu.make_async_copy(buf.at[NBUF - 1],
                                  acc_sh.at[pl.ds(s * ROWS_PT, EB)],
                                  isem).wait()
        plsc.subcore_barrier()

        def start_gather(p, b, i):
            pltpu.async_copy(y_hbm.at[srcb.at[p, b]], buf.at[i], gsem)

        def wait_gather(p, b, i):
            pltpu.make_async_copy(y_hbm.at[srcb.at[p, b]], buf.at[i],
                                  gsem).wait()

        def start_scatter(p, b, i):
            pltpu.async_copy(buf.at[i], acc_sh.at[dstb.at[p, b]], ssem,
                             add=True)

        def wait_scatter(p, b, i):
            pltpu.make_async_copy(buf.at[i], acc_sh.at[dstb.at[p, b]],
                                  ssem).wait()

        @pl.loop(0, NCHUNK)
        def _(cc):
            p = lax.rem(cc, 2)
            pn = lax.rem(cc + 1, 2)
            g0 = cc * CHUNK

            def ib(off):
                return lax.rem(g0 + off, NBUF)

            for b in range(CHUNK):
                wait_gather(p, b, ib(b))
                start_scatter(p, b, ib(b))

                if b == 0:
                    @pl.when(cc >= 1)
                    def _():
                        wait_scatter(pn, CHUNK - 1, ib(-1))
                else:
                    wait_scatter(p, b - 1, ib(b - 1))

                if b == 0:
                    @pl.when(cc + 1 < NCHUNK)
                    def _():
                        pltpu.async_copy(e_hbm.at[0, wid, cc + 1], srcb.at[pn],
                                         isem)
                        pltpu.async_copy(e_hbm.at[1, wid, cc + 1], dstb.at[pn],
                                         isem)
                if b == 3:
                    @pl.when(cc + 1 < NCHUNK)
                    def _():
                        pltpu.make_async_copy(e_hbm.at[0, wid, 0], srcb.at[pn],
                                              isem).wait()
                        pltpu.make_async_copy(e_hbm.at[1, wid, 0], dstb.at[pn],
                                              isem).wait()

                bn = b + NBUF - 1
                if bn < CHUNK:
                    start_gather(p, bn, ib(bn))
                else:
                    @pl.when(cc + 1 < NCHUNK)
                    def _():
                        start_gather(pn, bn - CHUNK, ib(bn))

        wait_scatter((NCHUNK - 1) % 2, CHUNK - 1, (NB - 1) % NBUF)
        plsc.subcore_barrier()

        GB = 64

        def round_idx(r):
            if r < 4:
                return tgt_v.at[r // 2, pl.ds((r % 2) * GB, GB)]
            q = r - 4
            return tgt_v.at[c, pl.ds(q * GB, GB)]

        def round_table(r):
            return acc_sh if r < 4 else y_hbm

        def round_out(r):
            if r < 4:
                return p_hbm.at[c, pl.ds(s * TPT + r * GB, GB)]
            q = r - 4
            return yt_hbm.at[pl.ds(s * TPT + c * 128 + q * GB, GB)]

        for r in range(6):
            pltpu.async_copy(round_table(r).at[round_idx(r)],
                             buf.at[r % NBUF], gsem).wait()
            pltpu.sync_copy(buf.at[r % NBUF], round_out(r))

        @pl.when(c == 0)
        def _():
            for j in range(2):
                pltpu.async_copy(dinv_hbm.at[tgt_v.at[j]],
                                 dt_v.at[pl.ds(j * 128, 128)], gsem).wait()
            pltpu.sync_copy(dt_v, dt_hbm.at[pl.ds(s * TPT, TPT)])

    return k(y, e5, tgt3d, dinv)


def _mlp_kernel(truck_x, P, yt, dt, W_truck, b_truck, b_gcn,
                W_comb, b_comb, W_out, b_out):
    BLK = 1024
    NCLS = 64

    def body(tx_ref, p_ref, yt_ref, dt_ref, wt_ref, bt_ref, bg_ref,
             wc_ref, bc_ref, wo_ref, bo_ref, out_ref):
        te = jnp.maximum(
            jnp.dot(tx_ref[...], wt_ref[...], preferred_element_type=jnp.float32)
            + bt_ref[...], 0.0)
        pp = p_ref[...]
        dt_col = dt_ref[...].reshape(BLK, 1)
        node = jnp.maximum(
            dt_col * (pp[0] + pp[1] + yt_ref[...]) + bg_ref[...], 0.0)
        wc = wc_ref[...]
        h = jnp.maximum(
            jnp.dot(te, wc[:D], preferred_element_type=jnp.float32)
            + jnp.dot(node, wc[D:], preferred_element_type=jnp.float32)
            + bc_ref[...], 0.0)
        out_ref[...] = (
            lax.dot_general(wo_ref[...], h, (((0,), (1,)), ((), ())),
                            preferred_element_type=jnp.float32)
            + bo_ref[...])

    full = lambda shape: pl.BlockSpec(shape, lambda i: tuple(0 for _ in shape))
    return pl.pallas_call(
        body,
        grid=(T // BLK,),
        in_specs=[
            pl.BlockSpec((BLK, 32), lambda i: (i, 0)),
            pl.BlockSpec((NC, BLK, D), lambda i: (0, i, 0)),
            pl.BlockSpec((BLK, D), lambda i: (i, 0)),
            pl.BlockSpec((BLK,), lambda i: (i,)),
            full((32, D)), full((1, D)), full((1, D)),
            full((2 * D, D)), full((1, D)),
            full((D, NCLS)), full((NCLS, 1)),
        ],
        out_specs=pl.BlockSpec((NCLS, BLK), lambda i: (0, i)),
        out_shape=jax.ShapeDtypeStruct((NCLS, T), jnp.float32),
    )(truck_x, P, yt, dt, W_truck, b_truck, b_gcn, W_comb, b_comb,
      W_out, b_out)


def kernel(x, edge_index, truck_x, cur_targets, W_gcn, b_gcn, W_truck, b_truck,
           W_comb, b_comb, W_out, b_out):
    ei = edge_index.astype(jnp.int32)
    pad_rows = N_NODES + (jnp.arange(EPAD - E, dtype=jnp.int32)
                          % (NPAD - N_NODES))
    pad2 = jnp.broadcast_to(pad_rows, (2, EPAD - E))
    e5 = jnp.concatenate([ei, pad2], axis=1).reshape(2, NW, NCHUNK, CHUNK, EB)
    tgt3d = cur_targets.astype(jnp.int32).reshape(NS, 2, 128)

    degp = _deg_kernel(e5)
    y, dinv = _xw_kernel(x, W_gcn, degp)
    P, yt, dt = _edge_kernel(y, e5, tgt3d, dinv)

    out_t = _mlp_kernel(
        truck_x, P, yt, dt,
        W_truck, b_truck.reshape(1, D), b_gcn.reshape(1, D),
        W_comb, b_comb.reshape(1, D),
        W_out, b_out.reshape(64, 1))
    return out_t.T
